# baseline (speedup 1.0000x reference)
import jax
import jax.numpy as jnp
from jax.experimental import pallas as pl
from jax.experimental.pallas import tpu as pltpu

IDIM = 64
K = 1000
KPAD = 1024
TNUM = 10
NITER = 10
CA = 385.0 / NITER
CB = 2.0 * 55.0 / NITER
BLK = 256


def _select_kernel(x_ref, valid_ref, e_ref, w_ref, b_ref, kb_ref,
                   p_out_ref, a_out_ref, inv_ref, ew_ref, acca_ref):
    i = pl.program_id(0)
    nsteps = pl.num_programs(0)

    @pl.when(i == 0)
    def _init():
        e = e_ref[...]
        norm2 = jnp.sum(e * e, axis=1, keepdims=True).T
        inv_ref[...] = jnp.where(norm2 > 0.0, 1.0 / jnp.sqrt(norm2), 0.0)
        ew_ref[...] = jax.lax.dot(e, w_ref[...],
                                  preferred_element_type=jnp.float32)
        acca_ref[...] = jnp.zeros_like(acca_ref)

    x = x_ref[...]
    s = jax.lax.dot_general(x, e_ref[...], (((1,), (1,)), ((), ())),
                            preferred_element_type=jnp.float32)
    s = s * inv_ref[...] + kb_ref[...]
    idx = jnp.argmax(s, axis=1)
    col = jax.lax.broadcasted_iota(jnp.int32, (BLK, KPAD), 1)
    onehot = (col == idx[:, None]).astype(jnp.float32)
    p = jax.lax.dot(onehot, ew_ref[...],
                    preferred_element_type=jnp.float32)
    p = p + b_ref[...]
    p_out_ref[...] = p

    v = valid_ref[...]
    acca_ref[...] += (p * p) * v

    @pl.when(i == nsteps - 1)
    def _fin():
        a_out_ref[...] = jnp.reshape(CA * jnp.sum(acca_ref[...]), (1, 1))


def _target_kernel(xso_ref, p_ref, valid_ref, out_ref, accv_ref):
    i = pl.program_id(0)
    nsteps = pl.num_programs(0)

    @pl.when(i == 0)
    def _init():
        accv_ref[...] = jnp.zeros_like(accv_ref)

    xo = xso_ref[...]
    t2 = (xo[:, 0:128] + xo[:, 128:256] + xo[:, 256:384]
          + xo[:, 384:512] + xo[:, 512:640])
    t = (t2[:, 0:IDIM] + t2[:, IDIM:2 * IDIM]) * (1.0 / TNUM)

    p = p_ref[...]
    v = valid_ref[...]
    accv_ref[...] += (t * t - CB * (p * t)) * v

    @pl.when(i == nsteps - 1)
    def _fin():
        out_ref[...] = jnp.reshape(jnp.sum(accv_ref[...]), (1, 1))


def _run(xs_pad_in, xs_pad_out, ilens, embed_weight, W_inf, b_inf,
         interpret=False):
    B, T, _ = xs_pad_in.shape
    N = B * T
    x = xs_pad_in.reshape(N, IDIM)
    xso = xs_pad_out.reshape(N, TNUM * IDIM)
    valid = (jnp.arange(T, dtype=jnp.int32)[None, :]
             < ilens[:, None].astype(jnp.int32)).astype(jnp.float32)
    valid = valid.reshape(N, 1)
    epad = jnp.zeros((KPAD, IDIM), jnp.float32).at[:K, :].set(embed_weight)
    kb = jnp.where(jnp.arange(KPAD)[None, :] < K, 0.0, -1e30
                   ).astype(jnp.float32)
    b2 = b_inf.reshape(1, IDIM)

    grid = (N // BLK,)
    p_all, a_part = pl.pallas_call(
        _select_kernel,
        grid=grid,
        in_specs=[
            pl.BlockSpec((BLK, IDIM), lambda i: (i, 0)),
            pl.BlockSpec((BLK, 1), lambda i: (i, 0)),
            pl.BlockSpec((KPAD, IDIM), lambda i: (0, 0)),
            pl.BlockSpec((IDIM, IDIM), lambda i: (0, 0)),
            pl.BlockSpec((1, IDIM), lambda i: (0, 0)),
            pl.BlockSpec((1, KPAD), lambda i: (0, 0)),
        ],
        out_specs=[
            pl.BlockSpec((BLK, IDIM), lambda i: (i, 0)),
            pl.BlockSpec((1, 1), lambda i: (0, 0)),
        ],
        out_shape=[
            jax.ShapeDtypeStruct((N, IDIM), jnp.float32),
            jax.ShapeDtypeStruct((1, 1), jnp.float32),
        ],
        scratch_shapes=[
            pltpu.VMEM((1, KPAD), jnp.float32),
            pltpu.VMEM((KPAD, IDIM), jnp.float32),
            pltpu.VMEM((BLK, IDIM), jnp.float32),
        ],
        interpret=interpret,
    )(x, valid, epad, W_inf, b2, kb)

    bc_part = pl.pallas_call(
        _target_kernel,
        grid=grid,
        in_specs=[
            pl.BlockSpec((BLK, TNUM * IDIM), lambda i: (i, 0)),
            pl.BlockSpec((BLK, IDIM), lambda i: (i, 0)),
            pl.BlockSpec((BLK, 1), lambda i: (i, 0)),
        ],
        out_specs=pl.BlockSpec((1, 1), lambda i: (0, 0)),
        out_shape=jax.ShapeDtypeStruct((1, 1), jnp.float32),
        scratch_shapes=[
            pltpu.VMEM((BLK, IDIM), jnp.float32),
        ],
        interpret=interpret,
    )(xso, p_all, valid)

    return (a_part + bc_part).reshape(())


def kernel(xs_pad_in, xs_pad_out, ilens, ys_pad, embed_weight, W_inf, b_inf):
    return _run(xs_pad_in, xs_pad_out, ilens, embed_weight, W_inf, b_inf)


# speedup vs baseline: 1.6826x; 1.6826x over previous
import jax
import jax.numpy as jnp
from jax.experimental import pallas as pl
from jax.experimental.pallas import tpu as pltpu

IDIM = 64
K = 1000
KPAD = 1024
TNUM = 10
NITER = 10
CA = 385.0 / NITER
CB = 2.0 * 55.0 / NITER
BLK = 256


def _vq_loss_kernel(x_ref, xso_ref, valid_ref, e_ref, w_ref, b_ref, kb_ref,
                    out_ref, inv_ref, ew_ref, accv_ref):
    i = pl.program_id(0)
    nsteps = pl.num_programs(0)

    @pl.when(i == 0)
    def _init():
        e = e_ref[...]
        norm2 = jnp.sum(e * e, axis=1, keepdims=True).T
        inv_ref[...] = jnp.where(norm2 > 0.0, 1.0 / jnp.sqrt(norm2), 0.0)
        ew_ref[...] = jax.lax.dot(e, w_ref[...],
                                  preferred_element_type=jnp.float32)
        accv_ref[...] = jnp.zeros_like(accv_ref)

    x = x_ref[0]
    s = jax.lax.dot_general(x, e_ref[...], (((1,), (1,)), ((), ())),
                            preferred_element_type=jnp.float32)
    s = s * inv_ref[...] + kb_ref[...]
    idx = jnp.argmax(s, axis=1)
    col = jax.lax.broadcasted_iota(jnp.int32, (BLK, KPAD), 1)
    onehot = (col == idx[:, None]).astype(jnp.float32)
    p = jax.lax.dot(onehot, ew_ref[...],
                    preferred_element_type=jnp.float32)
    p = p + b_ref[...]

    xo = xso_ref[0]
    t2 = (xo[:, 0:128] + xo[:, 128:256] + xo[:, 256:384]
          + xo[:, 384:512] + xo[:, 512:640])
    t = (t2[:, 0:IDIM] + t2[:, IDIM:2 * IDIM]) * (1.0 / TNUM)

    v = valid_ref[0]
    accv_ref[...] += (CA * (p * p) - CB * (p * t) + t * t) * v

    @pl.when(i == nsteps - 1)
    def _fin():
        out_ref[...] = jnp.reshape(jnp.sum(accv_ref[...]), (1, 1))


def _run(xs_pad_in, xs_pad_out, ilens, embed_weight, W_inf, b_inf,
         interpret=False):
    B, T, _ = xs_pad_in.shape
    N = B * T
    tb = T // BLK
    xso = xs_pad_out.reshape(B, T, TNUM * IDIM)
    valid = (jnp.arange(T, dtype=jnp.int32)[None, :, None]
             < ilens[:, None, None].astype(jnp.int32)).astype(jnp.float32)
    epad = jnp.zeros((KPAD, IDIM), jnp.float32).at[:K, :].set(embed_weight)
    kb = jnp.where(jnp.arange(KPAD)[None, :] < K, 0.0, -1e30
                   ).astype(jnp.float32)
    b2 = b_inf.reshape(1, IDIM)

    grid = (N // BLK,)
    out = pl.pallas_call(
        _vq_loss_kernel,
        grid=grid,
        in_specs=[
            pl.BlockSpec((1, BLK, IDIM), lambda i: (i // tb, i % tb, 0)),
            pl.BlockSpec((1, BLK, TNUM * IDIM),
                         lambda i: (i // tb, i % tb, 0)),
            pl.BlockSpec((1, BLK, 1), lambda i: (i // tb, i % tb, 0)),
            pl.BlockSpec((KPAD, IDIM), lambda i: (0, 0)),
            pl.BlockSpec((IDIM, IDIM), lambda i: (0, 0)),
            pl.BlockSpec((1, IDIM), lambda i: (0, 0)),
            pl.BlockSpec((1, KPAD), lambda i: (0, 0)),
        ],
        out_specs=pl.BlockSpec((1, 1), lambda i: (0, 0)),
        out_shape=jax.ShapeDtypeStruct((1, 1), jnp.float32),
        scratch_shapes=[
            pltpu.VMEM((1, KPAD), jnp.float32),
            pltpu.VMEM((KPAD, IDIM), jnp.float32),
            pltpu.VMEM((BLK, IDIM), jnp.float32),
        ],
        interpret=interpret,
    )(xs_pad_in, xso, valid, epad, W_inf, b2, kb)
    return out.reshape(())


def kernel(xs_pad_in, xs_pad_out, ilens, ys_pad, embed_weight, W_inf, b_inf):
    return _run(xs_pad_in, xs_pad_out, ilens, embed_weight, W_inf, b_inf)


# speedup vs baseline: 1.8545x; 1.1021x over previous
import jax
import jax.numpy as jnp
from jax.experimental import pallas as pl
from jax.experimental.pallas import tpu as pltpu

IDIM = 64
K = 1000
KPAD = 1024
TNUM = 10
NITER = 10
CA = 385.0 / NITER
CB = 2.0 * 55.0 / NITER
BLK = 512


def _vq_loss_kernel(x_ref, xso_ref, valid_ref, e_ref, w_ref, b_ref, kb_ref,
                    out_ref, inv_ref, ew_ref, accv_ref):
    i = pl.program_id(0)
    nsteps = pl.num_programs(0)

    @pl.when(i == 0)
    def _init():
        e = e_ref[...]
        norm2 = jnp.sum(e * e, axis=1, keepdims=True).T
        inv_ref[...] = jnp.where(norm2 > 0.0, 1.0 / jnp.sqrt(norm2), 0.0)
        ew_ref[...] = jax.lax.dot(e, w_ref[...],
                                  preferred_element_type=jnp.float32)
        accv_ref[...] = jnp.zeros_like(accv_ref)

    x = x_ref[0]
    s = jax.lax.dot_general(x, e_ref[...], (((1,), (1,)), ((), ())),
                            preferred_element_type=jnp.float32)
    s = s * inv_ref[...] + kb_ref[...]
    idx = jnp.argmax(s, axis=1)
    col = jax.lax.broadcasted_iota(jnp.int32, (BLK, KPAD), 1)
    onehot = (col == idx[:, None]).astype(jnp.float32)
    p = jax.lax.dot(onehot, ew_ref[...],
                    preferred_element_type=jnp.float32)
    p = p + b_ref[...]

    xo = xso_ref[0]
    t2 = (xo[:, 0:128] + xo[:, 128:256] + xo[:, 256:384]
          + xo[:, 384:512] + xo[:, 512:640])
    t = (t2[:, 0:IDIM] + t2[:, IDIM:2 * IDIM]) * (1.0 / TNUM)

    v = valid_ref[0]
    accv_ref[...] += (CA * (p * p) - CB * (p * t) + t * t) * v

    @pl.when(i == nsteps - 1)
    def _fin():
        out_ref[...] = jnp.reshape(jnp.sum(accv_ref[...]), (1, 1))


def _run(xs_pad_in, xs_pad_out, ilens, embed_weight, W_inf, b_inf,
         interpret=False):
    B, T, _ = xs_pad_in.shape
    N = B * T
    tb = T // BLK
    xso = xs_pad_out.reshape(B, T, TNUM * IDIM)
    valid = (jnp.arange(T, dtype=jnp.int32)[None, :, None]
             < ilens[:, None, None].astype(jnp.int32)).astype(jnp.float32)
    epad = jnp.zeros((KPAD, IDIM), jnp.float32).at[:K, :].set(embed_weight)
    kb = jnp.where(jnp.arange(KPAD)[None, :] < K, 0.0, -1e30
                   ).astype(jnp.float32)
    b2 = b_inf.reshape(1, IDIM)

    grid = (N // BLK,)
    out = pl.pallas_call(
        _vq_loss_kernel,
        grid=grid,
        in_specs=[
            pl.BlockSpec((1, BLK, IDIM), lambda i: (i // tb, i % tb, 0)),
            pl.BlockSpec((1, BLK, TNUM * IDIM),
                         lambda i: (i // tb, i % tb, 0)),
            pl.BlockSpec((1, BLK, 1), lambda i: (i // tb, i % tb, 0)),
            pl.BlockSpec((KPAD, IDIM), lambda i: (0, 0)),
            pl.BlockSpec((IDIM, IDIM), lambda i: (0, 0)),
            pl.BlockSpec((1, IDIM), lambda i: (0, 0)),
            pl.BlockSpec((1, KPAD), lambda i: (0, 0)),
        ],
        out_specs=pl.BlockSpec((1, 1), lambda i: (0, 0)),
        out_shape=jax.ShapeDtypeStruct((1, 1), jnp.float32),
        scratch_shapes=[
            pltpu.VMEM((1, KPAD), jnp.float32),
            pltpu.VMEM((KPAD, IDIM), jnp.float32),
            pltpu.VMEM((BLK, IDIM), jnp.float32),
        ],
        interpret=interpret,
    )(xs_pad_in, xso, valid, epad, W_inf, b2, kb)
    return out.reshape(())


def kernel(xs_pad_in, xs_pad_out, ilens, ys_pad, embed_weight, W_inf, b_inf):
    return _run(xs_pad_in, xs_pad_out, ilens, embed_weight, W_inf, b_inf)


# speedup vs baseline: 2.0897x; 1.1268x over previous
import jax
import jax.numpy as jnp
from jax.experimental import pallas as pl
from jax.experimental.pallas import tpu as pltpu

IDIM = 64
K = 1000
KPAD = 1024
TNUM = 10
NITER = 10
CA = 385.0 / NITER
CB = 2.0 * 55.0 / NITER
BLK = 512


def _vq_loss_kernel(x_ref, xso_ref, valid_ref, e_ref, w_ref, b_ref, kb_ref,
                    out_ref, inv_ref, ew_ref, accv_ref):
    i = pl.program_id(0)
    nsteps = pl.num_programs(0)

    @pl.when(i == 0)
    def _init():
        e = e_ref[...]
        norm2 = jnp.sum(e * e, axis=1, keepdims=True).T
        inv_ref[...] = jnp.where(norm2 > 0.0, 1.0 / jnp.sqrt(norm2), 0.0)
        ew_ref[:, 0:IDIM] = jax.lax.dot(e, w_ref[...],
                                        preferred_element_type=jnp.float32)
        ew_ref[:, IDIM:2 * IDIM] = jnp.ones((KPAD, IDIM), jnp.float32)
        accv_ref[...] = jnp.zeros_like(accv_ref)

    x = x_ref[0]
    s = jax.lax.dot_general(x, e_ref[...], (((1,), (1,)), ((), ())),
                            preferred_element_type=jnp.float32)
    s = s * inv_ref[...] + kb_ref[...]
    m = jnp.max(s, axis=1, keepdims=True)
    onehot = (s == m).astype(jnp.float32)
    p2 = jax.lax.dot(onehot, ew_ref[...],
                     preferred_element_type=jnp.float32)
    p = p2[:, 0:IDIM] / p2[:, IDIM:2 * IDIM] + b_ref[...]

    xo = xso_ref[0]
    t2 = (xo[:, 0:128] + xo[:, 128:256] + xo[:, 256:384]
          + xo[:, 384:512] + xo[:, 512:640])
    t = (t2[:, 0:IDIM] + t2[:, IDIM:2 * IDIM]) * (1.0 / TNUM)

    v = valid_ref[0]
    accv_ref[...] += (CA * (p * p) - CB * (p * t) + t * t) * v

    @pl.when(i == nsteps - 1)
    def _fin():
        out_ref[...] = jnp.reshape(jnp.sum(accv_ref[...]), (1, 1))


def _run(xs_pad_in, xs_pad_out, ilens, embed_weight, W_inf, b_inf,
         interpret=False):
    B, T, _ = xs_pad_in.shape
    N = B * T
    tb = T // BLK
    xso = xs_pad_out.reshape(B, T, TNUM * IDIM)
    valid = (jnp.arange(T, dtype=jnp.int32)[None, :, None]
             < ilens[:, None, None].astype(jnp.int32)).astype(jnp.float32)
    epad = jnp.zeros((KPAD, IDIM), jnp.float32).at[:K, :].set(embed_weight)
    kb = jnp.where(jnp.arange(KPAD)[None, :] < K, 0.0, -1e30
                   ).astype(jnp.float32)
    b2 = b_inf.reshape(1, IDIM)

    grid = (N // BLK,)
    out = pl.pallas_call(
        _vq_loss_kernel,
        grid=grid,
        in_specs=[
            pl.BlockSpec((1, BLK, IDIM), lambda i: (i // tb, i % tb, 0)),
            pl.BlockSpec((1, BLK, TNUM * IDIM),
                         lambda i: (i // tb, i % tb, 0)),
            pl.BlockSpec((1, BLK, 1), lambda i: (i // tb, i % tb, 0)),
            pl.BlockSpec((KPAD, IDIM), lambda i: (0, 0)),
            pl.BlockSpec((IDIM, IDIM), lambda i: (0, 0)),
            pl.BlockSpec((1, IDIM), lambda i: (0, 0)),
            pl.BlockSpec((1, KPAD), lambda i: (0, 0)),
        ],
        out_specs=pl.BlockSpec((1, 1), lambda i: (0, 0)),
        out_shape=jax.ShapeDtypeStruct((1, 1), jnp.float32),
        scratch_shapes=[
            pltpu.VMEM((1, KPAD), jnp.float32),
            pltpu.VMEM((KPAD, 2 * IDIM), jnp.float32),
            pltpu.VMEM((BLK, IDIM), jnp.float32),
        ],
        interpret=interpret,
    )(xs_pad_in, xso, valid, epad, W_inf, b2, kb)
    return out.reshape(())


def kernel(xs_pad_in, xs_pad_out, ilens, ys_pad, embed_weight, W_inf, b_inf):
    return _run(xs_pad_in, xs_pad_out, ilens, embed_weight, W_inf, b_inf)


# speedup vs baseline: 2.2631x; 1.0830x over previous
import jax
import jax.numpy as jnp
from jax.experimental import pallas as pl
from jax.experimental.pallas import tpu as pltpu

IDIM = 64
K = 1000
KPAD = 1024
TNUM = 10
NITER = 10
CA = 385.0 / NITER
CB = 2.0 * 55.0 / NITER
BLK = 1024


def _vq_loss_kernel(x_ref, xso_ref, valid_ref, e_ref, w_ref, b_ref, kb_ref,
                    out_ref, inv_ref, ew_ref, accv_ref):
    i = pl.program_id(0)
    nsteps = pl.num_programs(0)

    @pl.when(i == 0)
    def _init():
        e = e_ref[...]
        norm2 = jnp.sum(e * e, axis=1, keepdims=True).T
        inv_ref[...] = jnp.where(norm2 > 0.0, 1.0 / jnp.sqrt(norm2), 0.0)
        ew_ref[:, 0:IDIM] = jax.lax.dot(e, w_ref[...],
                                        preferred_element_type=jnp.float32)
        ew_ref[:, IDIM:2 * IDIM] = jnp.ones((KPAD, IDIM), jnp.float32)
        accv_ref[...] = jnp.zeros_like(accv_ref)

    x = x_ref[0]
    s = jax.lax.dot_general(x, e_ref[...], (((1,), (1,)), ((), ())),
                            preferred_element_type=jnp.float32)
    s = s * inv_ref[...] + kb_ref[...]
    m = jnp.max(s, axis=1, keepdims=True)
    onehot = (s == m).astype(jnp.float32)
    p2 = jax.lax.dot(onehot, ew_ref[...],
                     preferred_element_type=jnp.float32)
    p = p2[:, 0:IDIM] / p2[:, IDIM:2 * IDIM] + b_ref[...]

    xo = xso_ref[0]
    t2 = (xo[:, 0:128] + xo[:, 128:256] + xo[:, 256:384]
          + xo[:, 384:512] + xo[:, 512:640])
    t = (t2[:, 0:IDIM] + t2[:, IDIM:2 * IDIM]) * (1.0 / TNUM)

    v = valid_ref[0]
    accv_ref[...] += (CA * (p * p) - CB * (p * t) + t * t) * v

    @pl.when(i == nsteps - 1)
    def _fin():
        out_ref[...] = jnp.reshape(jnp.sum(accv_ref[...]), (1, 1))


def _run(xs_pad_in, xs_pad_out, ilens, embed_weight, W_inf, b_inf,
         interpret=False):
    B, T, _ = xs_pad_in.shape
    N = B * T
    tb = T // BLK
    xso = xs_pad_out.reshape(B, T, TNUM * IDIM)
    valid = (jnp.arange(T, dtype=jnp.int32)[None, :, None]
             < ilens[:, None, None].astype(jnp.int32)).astype(jnp.float32)
    epad = jnp.zeros((KPAD, IDIM), jnp.float32).at[:K, :].set(embed_weight)
    kb = jnp.where(jnp.arange(KPAD)[None, :] < K, 0.0, -1e30
                   ).astype(jnp.float32)
    b2 = b_inf.reshape(1, IDIM)

    grid = (N // BLK,)
    out = pl.pallas_call(
        _vq_loss_kernel,
        grid=grid,
        in_specs=[
            pl.BlockSpec((1, BLK, IDIM), lambda i: (i // tb, i % tb, 0)),
            pl.BlockSpec((1, BLK, TNUM * IDIM),
                         lambda i: (i // tb, i % tb, 0)),
            pl.BlockSpec((1, BLK, 1), lambda i: (i // tb, i % tb, 0)),
            pl.BlockSpec((KPAD, IDIM), lambda i: (0, 0)),
            pl.BlockSpec((IDIM, IDIM), lambda i: (0, 0)),
            pl.BlockSpec((1, IDIM), lambda i: (0, 0)),
            pl.BlockSpec((1, KPAD), lambda i: (0, 0)),
        ],
        out_specs=pl.BlockSpec((1, 1), lambda i: (0, 0)),
        out_shape=jax.ShapeDtypeStruct((1, 1), jnp.float32),
        scratch_shapes=[
            pltpu.VMEM((1, KPAD), jnp.float32),
            pltpu.VMEM((KPAD, 2 * IDIM), jnp.float32),
            pltpu.VMEM((BLK, IDIM), jnp.float32),
        ],
        interpret=interpret,
    )(xs_pad_in, xso, valid, epad, W_inf, b2, kb)
    return out.reshape(())


def kernel(xs_pad_in, xs_pad_out, ilens, ys_pad, embed_weight, W_inf, b_inf):
    return _run(xs_pad_in, xs_pad_out, ilens, embed_weight, W_inf, b_inf)


# speedup vs baseline: 2.3205x; 1.0254x over previous
import jax
import jax.numpy as jnp
from jax.experimental import pallas as pl
from jax.experimental.pallas import tpu as pltpu

IDIM = 64
K = 1000
KPAD = 1024
TNUM = 10
NITER = 10
CA = 385.0 / NITER
CB = 2.0 * 55.0 / NITER
BLK = 2048


def _vq_loss_kernel(x_ref, xso_ref, valid_ref, e_ref, w_ref, b_ref, kb_ref,
                    out_ref, inv_ref, ew_ref, accv_ref):
    i = pl.program_id(0)
    nsteps = pl.num_programs(0)

    @pl.when(i == 0)
    def _init():
        e = e_ref[...]
        norm2 = jnp.sum(e * e, axis=1, keepdims=True).T
        inv_ref[...] = jnp.where(norm2 > 0.0, 1.0 / jnp.sqrt(norm2), 0.0)
        ew_ref[:, 0:IDIM] = jax.lax.dot(e, w_ref[...],
                                        preferred_element_type=jnp.float32)
        ew_ref[:, IDIM:2 * IDIM] = jnp.ones((KPAD, IDIM), jnp.float32)
        accv_ref[...] = jnp.zeros_like(accv_ref)

    x = x_ref[0]
    s = jax.lax.dot_general(x, e_ref[...], (((1,), (1,)), ((), ())),
                            preferred_element_type=jnp.float32)
    s = s * inv_ref[...] + kb_ref[...]
    m = jnp.max(s, axis=1, keepdims=True)
    onehot = (s == m).astype(jnp.float32)
    p2 = jax.lax.dot(onehot, ew_ref[...],
                     preferred_element_type=jnp.float32)
    p = p2[:, 0:IDIM] / p2[:, IDIM:2 * IDIM] + b_ref[...]

    xo = xso_ref[0]
    t2 = (xo[:, 0:128] + xo[:, 128:256] + xo[:, 256:384]
          + xo[:, 384:512] + xo[:, 512:640])
    t = (t2[:, 0:IDIM] + t2[:, IDIM:2 * IDIM]) * (1.0 / TNUM)

    v = valid_ref[0]
    accv_ref[...] += (CA * (p * p) - CB * (p * t) + t * t) * v

    @pl.when(i == nsteps - 1)
    def _fin():
        out_ref[...] = jnp.reshape(jnp.sum(accv_ref[...]), (1, 1))


def _run(xs_pad_in, xs_pad_out, ilens, embed_weight, W_inf, b_inf,
         interpret=False):
    B, T, _ = xs_pad_in.shape
    N = B * T
    tb = T // BLK
    xso = xs_pad_out.reshape(B, T, TNUM * IDIM)
    valid = (jnp.arange(T, dtype=jnp.int32)[None, :, None]
             < ilens[:, None, None].astype(jnp.int32)).astype(jnp.float32)
    epad = jnp.zeros((KPAD, IDIM), jnp.float32).at[:K, :].set(embed_weight)
    kb = jnp.where(jnp.arange(KPAD)[None, :] < K, 0.0, -1e30
                   ).astype(jnp.float32)
    b2 = b_inf.reshape(1, IDIM)

    grid = (N // BLK,)
    out = pl.pallas_call(
        _vq_loss_kernel,
        grid=grid,
        in_specs=[
            pl.BlockSpec((1, BLK, IDIM), lambda i: (i // tb, i % tb, 0)),
            pl.BlockSpec((1, BLK, TNUM * IDIM),
                         lambda i: (i // tb, i % tb, 0)),
            pl.BlockSpec((1, BLK, 1), lambda i: (i // tb, i % tb, 0)),
            pl.BlockSpec((KPAD, IDIM), lambda i: (0, 0)),
            pl.BlockSpec((IDIM, IDIM), lambda i: (0, 0)),
            pl.BlockSpec((1, IDIM), lambda i: (0, 0)),
            pl.BlockSpec((1, KPAD), lambda i: (0, 0)),
        ],
        out_specs=pl.BlockSpec((1, 1), lambda i: (0, 0)),
        out_shape=jax.ShapeDtypeStruct((1, 1), jnp.float32),
        scratch_shapes=[
            pltpu.VMEM((1, KPAD), jnp.float32),
            pltpu.VMEM((KPAD, 2 * IDIM), jnp.float32),
            pltpu.VMEM((BLK, IDIM), jnp.float32),
        ],
        interpret=interpret,
    )(xs_pad_in, xso, valid, epad, W_inf, b2, kb)
    return out.reshape(())


def kernel(xs_pad_in, xs_pad_out, ilens, ys_pad, embed_weight, W_inf, b_inf):
    return _run(xs_pad_in, xs_pad_out, ilens, embed_weight, W_inf, b_inf)
